# baseline (speedup 1.0000x reference)
import jax
import jax.numpy as jnp
from jax.experimental import pallas as pl

NKPTS = 8
NFRM = 2
NNODES = NKPTS * NFRM
C0 = 512
BATCH = 1024
FEAT = 2048
TB = 256
TC = 1024

_F32 = jnp.float32
_BF16 = jnp.bfloat16


def _elu(v):
    return jnp.where(v > 0, v, jnp.exp(v) - 1.0)


def _dense_kernel(x_ref, w_ref, b_ref, o_ref):
    acc = jnp.dot(x_ref[...].astype(_BF16), w_ref[...].astype(_BF16),
                  preferred_element_type=_F32)
    o_ref[...] = (acc + b_ref[...]).astype(_BF16)


def _spiral_layer(H, Wv, bb, cin, act):
    Ws = [Wv[s * cin:(s + 1) * cin, :] for s in range(9)]
    F2 = NFRM * TB

    def dot(a, w):
        return jnp.dot(a, w, preferred_element_type=_F32)

    self_all = dot(H, Ws[0])
    time_all = dot(H, Ws[8])
    A = {j: dot(H[j * F2:(j + 1) * F2], Ws[j]) for j in range(1, NKPTS)}
    B = {j: dot(H[j * F2:(j + 1) * F2], Ws[j + 1]) for j in range(NKPTS - 1)}
    C = [None]
    acc = None
    for j in range(NKPTS - 1):
        acc = B[j] if acc is None else acc + B[j]
        C.append(acc)
    D = [None] * NKPTS
    acc = None
    for n in range(NKPTS - 2, -1, -1):
        acc = A[n + 1] if acc is None else acc + A[n + 1]
        D[n] = acc

    out_blocks = []
    for k in range(NKPTS):
        blk = self_all[k * F2:(k + 1) * F2] + bb
        if C[k] is not None:
            blk = blk + C[k]
        if D[k] is not None:
            blk = blk + D[k]
        t0 = time_all[(2 * k + 1) * TB:(2 * k + 2) * TB]
        t1 = time_all[(2 * k) * TB:(2 * k + 1) * TB]
        blk = blk + jnp.concatenate([t0, t1], axis=0)
        out_blocks.append((_elu(blk) if act else blk).astype(_BF16))
    return jnp.concatenate(out_blocks, axis=0)


def _stack_kernel(h_ref, w1_ref, b1_ref, w2_ref, b2_ref, w3_ref, b3_ref,
                  w4e_ref, b4_ref, o_ref):
    H = jnp.concatenate(
        [h_ref[:, (f * NKPTS + k) * C0:(f * NKPTS + k + 1) * C0]
         for k in range(NKPTS) for f in range(NFRM)], axis=0)
    H = _spiral_layer(H, w1_ref[...].astype(_BF16), b1_ref[...], 512, True)
    H = _spiral_layer(H, w2_ref[...].astype(_BF16), b2_ref[...], 512, True)
    H = _spiral_layer(H, w3_ref[...].astype(_BF16), b3_ref[...], 256, True)
    hcat = jnp.concatenate(
        [H[(2 * (n % NKPTS) + n // NKPTS) * TB:
           (2 * (n % NKPTS) + n // NKPTS) * TB + TB]
         for n in range(NNODES)], axis=1)
    o_ref[...] = (
        jnp.dot(hcat, w4e_ref[...].astype(_BF16), preferred_element_type=_F32)
        + b4_ref[...]
    )


def _expand_w4(W4):
    cin = 128
    zblk = jnp.zeros((cin, 3), W4.dtype)
    cols = []
    for n in range(NNODES):
        f, r = divmod(n, NKPTS)
        base = f * NKPTS
        rows = []
        for m in range(NNODES):
            if m == n:
                s = 0
            elif base <= m < base + NKPTS:
                j = m - base
                s = j + 1 if j < r else j
            elif m == (1 - f) * NKPTS + r:
                s = 8
            else:
                s = None
            rows.append(zblk if s is None else W4[s * cin:(s + 1) * cin, :])
        cols.append(jnp.concatenate(rows, axis=0))
    return jnp.concatenate(cols, axis=1)


def kernel(x, W0, b0, W1, b1, W2, b2, W3, b3, W4, b4):
    nb = BATCH // TB
    nc = (NNODES * C0) // TC

    h = pl.pallas_call(
        _dense_kernel,
        grid=(nc,),
        in_specs=[
            pl.BlockSpec((BATCH, FEAT), lambda c: (0, 0)),
            pl.BlockSpec((FEAT, TC), lambda c: (0, c)),
            pl.BlockSpec((1, TC), lambda c: (0, c)),
        ],
        out_specs=pl.BlockSpec((BATCH, TC), lambda c: (0, c)),
        out_shape=jax.ShapeDtypeStruct((BATCH, NNODES * C0), _BF16),
    )(x, W0, b0.reshape(1, -1))

    W4e = _expand_w4(W4)

    const = lambda b: (0, 0)
    out = pl.pallas_call(
        _stack_kernel,
        grid=(nb,),
        in_specs=[
            pl.BlockSpec((TB, NNODES * C0), lambda b: (b, 0)),
            pl.BlockSpec(W1.shape, const),
            pl.BlockSpec((1, 512), const),
            pl.BlockSpec(W2.shape, const),
            pl.BlockSpec((1, 256), const),
            pl.BlockSpec(W3.shape, const),
            pl.BlockSpec((1, 128), const),
            pl.BlockSpec((NNODES * 128, NNODES * 3), const),
            pl.BlockSpec((1, NNODES * 3), const),
        ],
        out_specs=pl.BlockSpec((TB, NNODES * 3), lambda b: (b, 0)),
        out_shape=jax.ShapeDtypeStruct((BATCH, NNODES * 3), _F32),
    )(h, W1, b1.reshape(1, -1), W2, b2.reshape(1, -1), W3,
      b3.reshape(1, -1), W4e, jnp.tile(b4, NNODES).reshape(1, -1))

    return out.reshape(BATCH, NNODES, 3)
